# baseline (speedup 1.0000x reference)
import functools

import jax
import jax.numpy as jnp
from jax import lax
from jax.experimental import pallas as pl
from jax.experimental.pallas import tpu as pltpu
from jax.experimental.pallas import tpu_sc as plsc

_IOU_THRESHOLD = 0.5
_K = 300
_K1 = _K + 1
_KP = 384
_INT32_MAX = 2147483647


def _take16(v, idx):
    dnums = lax.GatherDimensionNumbers(
        offset_dims=(), collapsed_slice_dims=(0,), start_index_map=(0,))
    return lax.gather(v, idx[:, None], dnums, slice_sizes=(1,),
                      mode=lax.GatherScatterMode.PROMISE_IN_BOUNDS)


def _nms_body(n_boxes, np_pad, n_batches,
              sc_hbm, x1_hbm, y1_hbm, x2_hbm, y2_hbm, cls_hbm,
              sel_o, osc_o, ox1_o, oy1_o, ox2_o, oy2_o, ocl_o, meta_o,
              sc0, sx1, sy1, sx2, sy2, l1, l2,
              kx1, ky1, kx2, ky2, kar, ksel, ksc, kcl, gidx,
              meta_v, bad_v, st_s, sem):
    cid = lax.axis_index("c")
    sid = lax.axis_index("s")
    b = sid * 2 + cid
    i16 = lax.broadcasted_iota(jnp.int32, (16,), 0)
    lane0 = i16 == 0
    neg_inf = jnp.float32(-jnp.inf)
    nb1 = np_pad // 16
    nb2 = nb1 // 16

    def smax(v):
        for sh in (8, 4, 2, 1):
            v = jnp.maximum(v, _take16(v, (i16 + sh) & 15))
        return v

    def smin(v):
        for sh in (8, 4, 2, 1):
            v = jnp.minimum(v, _take16(v, (i16 + sh) & 15))
        return v

    def put(ref, pos, val):
        w = pl.ds(pos, 16)
        ref[w] = jnp.where(lane0, val, ref[w])

    @pl.when(b < n_batches)
    def _run():
        base = b * n_boxes
        zf = jnp.zeros((16,), jnp.float32)
        ninf = jnp.full((16,), neg_inf)

        cps = (pltpu.async_copy(sc_hbm.at[pl.ds(base, n_boxes)],
                                sc0.at[pl.ds(0, n_boxes)], sem),
               pltpu.async_copy(x1_hbm.at[pl.ds(base, n_boxes)],
                                sx1.at[pl.ds(0, n_boxes)], sem),
               pltpu.async_copy(y1_hbm.at[pl.ds(base, n_boxes)],
                                sy1.at[pl.ds(0, n_boxes)], sem),
               pltpu.async_copy(x2_hbm.at[pl.ds(base, n_boxes)],
                                sx2.at[pl.ds(0, n_boxes)], sem),
               pltpu.async_copy(y2_hbm.at[pl.ds(base, n_boxes)],
                                sy2.at[pl.ds(0, n_boxes)], sem))

        for j in range(_KP // 16):
            s = pl.ds(j * 16, 16)
            kx1[s] = zf
            ky1[s] = zf
            kx2[s] = zf
            ky2[s] = zf
            kar[s] = zf
            ksel[s] = jnp.full((16,), -1, jnp.int32)
            ksc[s] = zf
        for cp in cps:
            cp.wait()
        for j in range((np_pad - n_boxes) // 16):
            sc0[pl.ds(n_boxes + j * 16, 16)] = ninf

        def b1_build(jk, tok):
            v = sc0[pl.ds(jk * 16, 16)]
            put(l1, jk, smax(v)[0])
            return tok

        lax.fori_loop(0, nb1, b1_build, jnp.int32(0))

        def b2_build(jk, tok):
            v = l1[pl.ds(jk * 16, 16)]
            put(l2, jk, smax(v)[0])
            return tok

        lax.fori_loop(0, nb2, b2_build, jnp.int32(0))

        st_s[0] = 0
        st_s[1] = 0

        big = jnp.full((16,), 9999, jnp.int32)

        def examine(_, tok):
            count = st_s[0]
            active = (count < _K1) & (st_s[1] == 0)

            @pl.when(active)
            def _one():
                vs = [l2[pl.ds(16 * j, 16)] for j in range(nb2 // 16)]
                m = vs[0]
                for v in vs[1:]:
                    m = jnp.maximum(m, v)
                gm = smax(m)[0]

                @pl.when(gm == neg_inf)
                def _done():
                    st_s[1] = 1

                @pl.when(gm > neg_inf)
                def _pick():
                    cand = big
                    for j, v in enumerate(vs):
                        cand = jnp.minimum(
                            cand, jnp.where(v == gm, i16 + 16 * j, 9999))
                    b2i = smin(cand)[0]
                    w1 = l1[pl.ds(b2i * 16, 16)]
                    j1 = smin(jnp.where(w1 == gm, i16, 9999))[0]
                    b1i = b2i * 16 + j1
                    s0 = sc0[pl.ds(b1i * 16, 16)]
                    l0 = smin(jnp.where(s0 == gm, i16, 9999))[0]
                    idx = b1i * 16 + l0

                    iw = pl.ds(idx, 16)
                    x1c = sx1[iw][0]
                    y1c = sy1[iw][0]
                    x2c = sx2[iw][0]
                    y2c = sy2[iw][0]
                    ac = (x2c - x1c) * (y2c - y1c)

                    nkc = (count + 63) // 64
                    bad_v[...] = zf

                    def kchunk(jk, tok2):
                        worst = zf
                        for g in range(4):
                            ks = pl.ds(jk * 64 + g * 16, 16)
                            xx1 = jnp.maximum(kx1[ks], x1c)
                            yy1 = jnp.maximum(ky1[ks], y1c)
                            xx2 = jnp.minimum(kx2[ks], x2c)
                            yy2 = jnp.minimum(ky2[ks], y2c)
                            inter = (jnp.maximum(xx2 - xx1, 0.0) *
                                     jnp.maximum(yy2 - yy1, 0.0))
                            iou = inter / (kar[ks] + ac - inter + 1e-9)
                            worst = jnp.maximum(worst, iou)
                        bad_v[...] = jnp.maximum(bad_v[...], worst)
                        return tok2

                    lax.fori_loop(0, nkc, kchunk, jnp.int32(0))
                    wv = smax(bad_v[...])
                    keep = wv[0] <= _IOU_THRESHOLD

                    @pl.when(keep & (count < _K))
                    def _store():
                        put(kx1, count, x1c)
                        put(ky1, count, y1c)
                        put(kx2, count, x2c)
                        put(ky2, count, y2c)
                        put(kar, count, ac)
                        put(ksel, count, idx)
                        put(ksc, count, gm)

                    st_s[0] = count + keep.astype(jnp.int32)

                    s0n = jnp.where(i16 == l0, neg_inf, s0)
                    sc0[pl.ds(b1i * 16, 16)] = s0n
                    put(l1, b1i, smax(s0n)[0])
                    w1n = l1[pl.ds(b2i * 16, 16)]
                    put(l2, b2i, smax(w1n)[0])

            return tok

        def outer(_, tok):
            @pl.when((st_s[0] < _K1) & (st_s[1] == 0))
            def _block():
                lax.fori_loop(0, 128, examine, jnp.int32(0))
            return tok

        lax.fori_loop(0, np_pad // 128, outer, jnp.int32(0))

        count = st_s[0]
        meta_v[...] = jnp.where(lane0, count, 0)

        for j in range(_KP // 16):
            s = pl.ds(j * 16, 16)
            gidx[s] = jnp.maximum(ksel[s], 0) + base
        gcps = []
        for j in range(_KP // 128):
            row = gidx.at[pl.ds(j * 128, 128)]
            gcps.append(pltpu.async_copy(cls_hbm.at[row],
                                         kcl.at[pl.ds(j * 128, 128)], sem))
        for cp in gcps:
            cp.wait()

        pltpu.sync_copy(ksel, sel_o.at[b])
        pltpu.sync_copy(ksc, osc_o.at[b])
        pltpu.sync_copy(kx1, ox1_o.at[b])
        pltpu.sync_copy(ky1, oy1_o.at[b])
        pltpu.sync_copy(kx2, ox2_o.at[b])
        pltpu.sync_copy(ky2, oy2_o.at[b])
        pltpu.sync_copy(kcl, ocl_o.at[b])
        pltpu.sync_copy(meta_v, meta_o.at[b])


def kernel(scores, boxes, classes):
    B, N = scores.shape
    Np = ((N + 2047) // 2048) * 2048

    scf = scores.reshape(-1)
    x1f = boxes[:, :, 0].reshape(-1)
    y1f = boxes[:, :, 1].reshape(-1)
    x2f = boxes[:, :, 2].reshape(-1)
    y2f = boxes[:, :, 3].reshape(-1)
    clsf = classes.reshape(-1)

    mesh = plsc.VectorSubcoreMesh(core_axis_name="c", subcore_axis_name="s")
    out_type = [
        jax.ShapeDtypeStruct((B, _KP), jnp.int32),
        jax.ShapeDtypeStruct((B, _KP), jnp.float32),
        jax.ShapeDtypeStruct((B, _KP), jnp.float32),
        jax.ShapeDtypeStruct((B, _KP), jnp.float32),
        jax.ShapeDtypeStruct((B, _KP), jnp.float32),
        jax.ShapeDtypeStruct((B, _KP), jnp.float32),
        jax.ShapeDtypeStruct((B, _KP), jnp.int32),
        jax.ShapeDtypeStruct((B, 16), jnp.int32),
    ]
    scratch_types = [
        pltpu.VMEM((Np + 16,), jnp.float32),
        pltpu.VMEM((Np + 16,), jnp.float32),
        pltpu.VMEM((Np + 16,), jnp.float32),
        pltpu.VMEM((Np + 16,), jnp.float32),
        pltpu.VMEM((Np + 16,), jnp.float32),
        pltpu.VMEM((Np // 16 + 16,), jnp.float32),
        pltpu.VMEM((Np // 256 + 16,), jnp.float32),
        pltpu.VMEM((_KP,), jnp.float32),
        pltpu.VMEM((_KP,), jnp.float32),
        pltpu.VMEM((_KP,), jnp.float32),
        pltpu.VMEM((_KP,), jnp.float32),
        pltpu.VMEM((_KP,), jnp.float32),
        pltpu.VMEM((_KP,), jnp.int32),
        pltpu.VMEM((_KP,), jnp.float32),
        pltpu.VMEM((_KP,), jnp.int32),
        pltpu.VMEM((_KP,), jnp.int32),
        pltpu.VMEM((16,), jnp.int32),
        pltpu.VMEM((16,), jnp.float32),
        pltpu.SMEM((2,), jnp.int32),
        pltpu.SemaphoreType.DMA,
    ]
    fn = pl.kernel(
        functools.partial(_nms_body, N, Np, B),
        out_type=out_type,
        mesh=mesh,
        scratch_types=scratch_types,
    )
    sel_p, osc, ox1, oy1, ox2, oy2, ocl, meta = fn(
        scf, x1f, y1f, x2f, y2f, clsf)

    sel = sel_p[:, :_K]
    count = meta[:, 0]
    overflow = count > _K
    count = jnp.minimum(count, _K)
    eff = jnp.where(overflow, jnp.int32(_K - 1), count)
    m = jnp.arange(_K, dtype=jnp.int32)[None, :] < eff[:, None]
    out_scores = jnp.where(m, osc[:, :_K], 0.0)
    out_boxes = jnp.where(
        m[:, :, None],
        jnp.stack([ox1[:, :_K], oy1[:, :_K], ox2[:, :_K], oy2[:, :_K]],
                  axis=-1),
        0.0)
    out_classes = jnp.where(m, ocl[:, :_K], jnp.int32(_INT32_MAX))
    true_max = jnp.where(overflow, jnp.int32(-1), count).astype(jnp.int32)
    return (sel, out_scores, out_boxes, out_classes, true_max)
